# Initial kernel scaffold; baseline (speedup 1.0000x reference)
import jax
import jax.numpy as jnp
from jax.experimental import pallas as pl


def kernel(x, edge_index, batch, params):
    raise NotImplementedError("write your pallas kernel here")


# speedup vs baseline: 17.6175x; 17.6175x over previous
import functools

import jax
import jax.numpy as jnp
from jax import lax
from jax.experimental import pallas as pl
from jax.experimental.pallas import tpu as pltpu
from jax.experimental.pallas import tpu_sc as plsc

N = 10000
E = 320000
D = 128
G = 64

NC = 2
NS = 16
DH = D // NC
EPT = E // NS
BB = 80
NB = EPT // BB

BLK = 1000
NBLK = N // BLK

_F32 = jnp.float32
_HIGH = jax.lax.Precision.HIGHEST


def _prep_body(use_bn, *refs):
    if use_bn:
        (hin, stats, g, b, W, asrc, adst, h_out, as_out, ad_out, M_out,
         msc) = refs
    else:
        hin, W, asrc, adst, h_out, as_out, ad_out, M_out, msc = refs
    i = pl.program_id(0)
    xb = hin[...]
    if use_bn:
        mu = stats[0:1, :] * (1.0 / N)
        var = stats[1:2, :] * (1.0 / N) - mu * mu
        xb = (xb - mu) * lax.rsqrt(var + 1e-5) * g[...] + b[...]
        xb = jnp.maximum(xb, 0.0)
    h = jnp.dot(xb, W[...], preferred_element_type=_F32, precision=_HIGH)
    h_out[0] = h[:, :DH]
    h_out[1] = h[:, DH:]
    a_s = jnp.sum(h * asrc[...], axis=1, keepdims=True)
    a_d = jnp.sum(h * adst[...], axis=1, keepdims=True)
    as_out[...] = a_s
    ad_out[...] = a_d

    @pl.when(i == 0)
    def _():
        msc[0] = -jnp.inf
        msc[1] = -jnp.inf

    msc[0] = jnp.maximum(msc[0], jnp.max(a_s))
    msc[1] = jnp.maximum(msc[1], jnp.max(a_d))
    mm = msc[0] + msc[1]
    M = jnp.where(mm >= 0, mm, 0.2 * mm)
    M_out[...] = jnp.full((1, D), M, _F32)


def _make_prep(use_bn):
    full = lambda i: (0, 0)
    in_specs = [pl.BlockSpec((BLK, D), lambda i: (i, 0))]
    if use_bn:
        in_specs += [pl.BlockSpec((2, D), full), pl.BlockSpec((1, D), full),
                     pl.BlockSpec((1, D), full)]
    in_specs += [pl.BlockSpec((D, D), full), pl.BlockSpec((1, D), full),
                 pl.BlockSpec((1, D), full)]
    return pl.pallas_call(
        functools.partial(_prep_body, use_bn),
        grid=(NBLK,),
        in_specs=in_specs,
        out_specs=[pl.BlockSpec((NC, BLK, DH), lambda i: (0, i, 0)),
                   pl.BlockSpec((BLK, 1), lambda i: (i, 0)),
                   pl.BlockSpec((BLK, 1), lambda i: (i, 0)),
                   pl.BlockSpec((1, D), full)],
        out_shape=[jax.ShapeDtypeStruct((NC, N, DH), _F32),
                   jax.ShapeDtypeStruct((N, 1), _F32),
                   jax.ShapeDtypeStruct((N, 1), _F32),
                   jax.ShapeDtypeStruct((1, D), _F32)],
        scratch_shapes=[pltpu.SMEM((2,), _F32)],
    )


_prep0 = _make_prep(False)
_prep1 = _make_prep(True)


def _stats_body(num, z, b, hpre_out, stats_out, acc):
    i = pl.program_id(0)
    s = jnp.concatenate([num[0], num[1]], axis=1)
    zz = z[0]
    hp = s / (zz + 1e-16) + b[...]
    hpre_out[...] = hp

    @pl.when(i == 0)
    def _():
        acc[...] = jnp.zeros((2, D), _F32)

    acc[0:1, :] = acc[0:1, :] + jnp.sum(hp, axis=0, keepdims=True)
    acc[1:2, :] = acc[1:2, :] + jnp.sum(hp * hp, axis=0, keepdims=True)
    stats_out[...] = acc[...]


_stats_call = pl.pallas_call(
    _stats_body,
    grid=(NBLK,),
    in_specs=[pl.BlockSpec((NC, BLK, DH), lambda i: (0, i, 0)),
              pl.BlockSpec((NC, BLK, 1), lambda i: (0, i, 0)),
              pl.BlockSpec((1, D), lambda i: (0, 0))],
    out_specs=[pl.BlockSpec((BLK, D), lambda i: (i, 0)),
               pl.BlockSpec((2, D), lambda i: (0, 0))],
    out_shape=[jax.ShapeDtypeStruct((N, D), _F32),
               jax.ShapeDtypeStruct((2, D), _F32)],
    scratch_shapes=[pltpu.VMEM((2, D), _F32)],
)


def _pool_body(hpre, stats, g, b, bat, l1W, l1b, l2W, l2b, out, seg, cnt):
    i = pl.program_id(0)
    mu = stats[0:1, :] * (1.0 / N)
    var = stats[1:2, :] * (1.0 / N) - mu * mu
    h = jnp.maximum((hpre[...] - mu) * lax.rsqrt(var + 1e-5) * g[...] + b[...],
                    0.0)
    ids = lax.broadcasted_iota(jnp.int32, (BLK, G), 1)
    oh = (ids == bat[...]).astype(_F32)

    @pl.when(i == 0)
    def _():
        seg[...] = jnp.zeros((G, D), _F32)
        cnt[...] = jnp.zeros((G, 1), _F32)

    seg[...] = seg[...] + lax.dot_general(
        oh, h, (((0,), (0,)), ((), ())), preferred_element_type=_F32,
        precision=_HIGH)
    cnt[...] = cnt[...] + jnp.sum(oh, axis=0).reshape(G, 1)
    gf = seg[...] / jnp.maximum(cnt[...], 1.0)
    a1 = jnp.maximum(
        jnp.dot(gf, l1W[...], preferred_element_type=_F32, precision=_HIGH)
        + l1b[...], 0.0)
    out[...] = (jnp.dot(a1, l2W[...], preferred_element_type=_F32,
                        precision=_HIGH) + l2b[...])


_pool_call = pl.pallas_call(
    _pool_body,
    grid=(NBLK,),
    in_specs=[pl.BlockSpec((BLK, D), lambda i: (i, 0)),
              pl.BlockSpec((2, D), lambda i: (0, 0)),
              pl.BlockSpec((1, D), lambda i: (0, 0)),
              pl.BlockSpec((1, D), lambda i: (0, 0)),
              pl.BlockSpec((BLK, 1), lambda i: (i, 0)),
              pl.BlockSpec((D, D), lambda i: (0, 0)),
              pl.BlockSpec((1, D), lambda i: (0, 0)),
              pl.BlockSpec((D, 1), lambda i: (0, 0)),
              pl.BlockSpec((1, 1), lambda i: (0, 0))],
    out_specs=[pl.BlockSpec((G, 1), lambda i: (0, 0))],
    out_shape=[jax.ShapeDtypeStruct((G, 1), _F32)],
    scratch_shapes=[pltpu.VMEM((G, D), _F32), pltpu.VMEM((G, 1), _F32)],
)


NP = 10240
_R = NP // NS
_RC = 128


def _edge_body(h_hbm, as_hbm, ad_hbm, M_hbm, src_hbm, dst_hbm,
               num_out, z_out,
               srcb, dstb, asb, adb, Mb, rows, wb, zbuf, zb, num_sh, z_sh,
               sem):
    cid = lax.axis_index("c")
    sid = lax.axis_index("s")

    pltpu.sync_copy(src_hbm.at[sid], srcb)
    pltpu.sync_copy(dst_hbm.at[sid], dstb)
    pltpu.sync_copy(as_hbm, asb)
    pltpu.sync_copy(ad_hbm, adb)
    pltpu.sync_copy(M_hbm, Mb)

    def _zrow(i, c):
        for q in range(DH // 16):
            zbuf[i, pl.ds(q * 16, 16)] = jnp.zeros((16,), _F32)
        return c

    lax.fori_loop(0, _RC, _zrow, 0)
    for k in range(_R // _RC):
        pltpu.sync_copy(zbuf, num_sh.at[pl.ds(sid * _R + k * _RC, _RC)])

    def _zv(i, c):
        zb[pl.ds(i * 16, 16)] = jnp.zeros((16,), _F32)
        return c

    lax.fori_loop(0, 40, _zv, 0)
    pltpu.sync_copy(zb, z_sh.at[pl.ds(sid * _R, _R)])

    plsc.subcore_barrier()

    Mv = Mb[0, pl.ds(0, 16)]

    def _batch(bi, c):
        for gidx in range(BB // 16):
            srcv = srcb[bi, pl.ds(gidx * 16, 16)]
            dstv = dstb[bi, pl.ds(gidx * 16, 16)]
            e = plsc.load_gather(asb, [srcv]) + plsc.load_gather(adb, [dstv])
            e = jnp.where(e >= 0, e, 0.2 * e) - Mv
            wb[pl.ds(gidx * 16, 16)] = jnp.exp(e)
        pltpu.async_copy(h_hbm.at[cid].at[srcb.at[bi]], rows, sem).wait()

        def _srow(j, c2):
            wsp = plsc.load_gather(wb, [jnp.full((16,), j, jnp.int32)])
            for q in range(DH // 16):
                rows[j, pl.ds(q * 16, 16)] = rows[j, pl.ds(q * 16, 16)] * wsp
            return c2

        lax.fori_loop(0, BB, _srow, 0)
        pltpu.sync_copy(rows, num_sh.at[dstb.at[bi]], add=True)
        pltpu.sync_copy(wb, z_sh.at[dstb.at[bi]], add=True)
        return c

    lax.fori_loop(0, NB, _batch, 0)
    plsc.subcore_barrier()

    for k in range(_R // _RC):
        sl = pl.ds(sid * _R + k * _RC, _RC)
        pltpu.sync_copy(num_sh.at[sl], num_out.at[cid].at[sl])

    zsl = pl.ds(sid * _R, _R)
    pltpu.sync_copy(z_sh.at[zsl], z_out.at[cid].at[zsl])


_edge_call = pl.kernel(
    _edge_body,
    out_type=[jax.ShapeDtypeStruct((NC, NP, DH), _F32),
              jax.ShapeDtypeStruct((NC, NP), _F32)],
    mesh=plsc.VectorSubcoreMesh(core_axis_name="c", subcore_axis_name="s",
                                num_cores=NC, num_subcores=NS),
    scratch_types=[
        pltpu.VMEM((NB, BB), jnp.int32),
        pltpu.VMEM((NB, BB), jnp.int32),
        pltpu.VMEM((N,), _F32),
        pltpu.VMEM((N,), _F32),
        pltpu.VMEM((1, D), _F32),
        pltpu.VMEM((BB, DH), _F32),
        pltpu.VMEM((BB,), _F32),
        pltpu.VMEM((_RC, DH), _F32),
        pltpu.VMEM((_R,), _F32),
        pltpu.VMEM_SHARED((NP, DH), _F32),
        pltpu.VMEM_SHARED((NP,), _F32),
        pltpu.SemaphoreType.DMA,
    ],
    compiler_params=pltpu.CompilerParams(needs_layout_passes=False,
                                         use_tc_tiling_on_sc=False),
)


def kernel(x, edge_index, batch, params):
    src3 = edge_index[0].reshape(NS, NB, BB)
    dst3 = edge_index[1].reshape(NS, NB, BB)
    bat2 = batch.reshape(N, 1)

    def row(v):
        return v.reshape(1, D)

    def layer(hin, stats, bn_prev, gat):
        if stats is None:
            h, a_s, a_d, M = _prep0(hin, gat["W"], row(gat["a_src"]),
                                    row(gat["a_dst"]))
        else:
            h, a_s, a_d, M = _prep1(hin, stats, row(bn_prev["g"]),
                                    row(bn_prev["b"]), gat["W"],
                                    row(gat["a_src"]), row(gat["a_dst"]))
        num, z = _edge_call(h, a_s.reshape(N), a_d.reshape(N), M, src3, dst3)
        hpre, st = _stats_call(num, z.reshape(NC, NP, 1), row(gat["b"]))
        return hpre, st

    hpre, st = layer(x, None, None, params["gat1"])
    hpre, st = layer(hpre, st, params["bn1"], params["gat2"])
    hpre, st = layer(hpre, st, params["bn2"], params["gat3"])

    (out,) = _pool_call(hpre, st, row(params["bn3"]["g"]),
                        row(params["bn3"]["b"]), bat2,
                        params["lin1_W"], row(params["lin1_b"]),
                        params["lin2_W"], params["lin2_b"].reshape(1, 1))
    return out


# speedup vs baseline: 18.9294x; 1.0745x over previous
import functools

import jax
import jax.numpy as jnp
from jax import lax
from jax.experimental import pallas as pl
from jax.experimental.pallas import tpu as pltpu
from jax.experimental.pallas import tpu_sc as plsc

N = 10000
E = 320000
D = 128
G = 64

NC = 2
NS = 16
DH = D // NC
EPT = E // NS
BB = 80
NB = EPT // BB

BLK = 1000
NBLK = N // BLK

_F32 = jnp.float32


def _prep_body(use_bn, *refs):
    if use_bn:
        (hin, stats, g, b, W, asrc, adst, h_out, as_out, ad_out, M_out,
         msc) = refs
    else:
        hin, W, asrc, adst, h_out, as_out, ad_out, M_out, msc = refs
    i = pl.program_id(0)
    xb = hin[...]
    if use_bn:
        mu = stats[0:1, :] * (1.0 / N)
        var = stats[1:2, :] * (1.0 / N)
        xb = (xb - mu) * lax.rsqrt(var + 1e-5) * g[...] + b[...]
        xb = jnp.maximum(xb, 0.0)
    h = jnp.dot(xb, W[...], preferred_element_type=_F32)
    h_out[0] = h[:, :DH]
    h_out[1] = h[:, DH:]
    a_s = jnp.sum(h * asrc[...], axis=1, keepdims=True)
    a_d = jnp.sum(h * adst[...], axis=1, keepdims=True)
    as_out[...] = a_s
    ad_out[...] = a_d

    @pl.when(i == 0)
    def _():
        msc[0] = -jnp.inf
        msc[1] = -jnp.inf

    msc[0] = jnp.maximum(msc[0], jnp.max(a_s))
    msc[1] = jnp.maximum(msc[1], jnp.max(a_d))
    mm = msc[0] + msc[1]
    M = jnp.where(mm >= 0, mm, 0.2 * mm)
    M_out[...] = jnp.full((1, D), M, _F32)


def _make_prep(use_bn):
    full = lambda i: (0, 0)
    in_specs = [pl.BlockSpec((BLK, D), lambda i: (i, 0))]
    if use_bn:
        in_specs += [pl.BlockSpec((2, D), full), pl.BlockSpec((1, D), full),
                     pl.BlockSpec((1, D), full)]
    in_specs += [pl.BlockSpec((D, D), full), pl.BlockSpec((1, D), full),
                 pl.BlockSpec((1, D), full)]
    return pl.pallas_call(
        functools.partial(_prep_body, use_bn),
        grid=(NBLK,),
        in_specs=in_specs,
        out_specs=[pl.BlockSpec((NC, BLK, DH), lambda i: (0, i, 0)),
                   pl.BlockSpec((BLK, 1), lambda i: (i, 0)),
                   pl.BlockSpec((BLK, 1), lambda i: (i, 0)),
                   pl.BlockSpec((1, D), full)],
        out_shape=[jax.ShapeDtypeStruct((NC, N, DH), _F32),
                   jax.ShapeDtypeStruct((N, 1), _F32),
                   jax.ShapeDtypeStruct((N, 1), _F32),
                   jax.ShapeDtypeStruct((1, D), _F32)],
        scratch_shapes=[pltpu.SMEM((2,), _F32)],
    )


_prep0 = _make_prep(False)
_prep1 = _make_prep(True)


def _stats_body(num, z, b, hpre_out, sum_out, acc):
    i = pl.program_id(0)
    s = jnp.concatenate([num[0], num[1]], axis=1)
    zz = z[0]
    hp = s / (zz + 1e-16) + b[...]
    hpre_out[...] = hp

    @pl.when(i == 0)
    def _():
        acc[...] = jnp.zeros((1, D), _F32)

    acc[...] = acc[...] + jnp.sum(hp, axis=0, keepdims=True)
    sum_out[...] = acc[...]


_stats_call = pl.pallas_call(
    _stats_body,
    grid=(NBLK,),
    in_specs=[pl.BlockSpec((NC, BLK, DH), lambda i: (0, i, 0)),
              pl.BlockSpec((NC, BLK, 1), lambda i: (0, i, 0)),
              pl.BlockSpec((1, D), lambda i: (0, 0))],
    out_specs=[pl.BlockSpec((BLK, D), lambda i: (i, 0)),
               pl.BlockSpec((1, D), lambda i: (0, 0))],
    out_shape=[jax.ShapeDtypeStruct((N, D), _F32),
               jax.ShapeDtypeStruct((1, D), _F32)],
    scratch_shapes=[pltpu.VMEM((1, D), _F32)],
)


def _var_body(hpre, csum, var_out, acc):
    i = pl.program_id(0)
    mu = csum[...] * (1.0 / N)
    d = hpre[...] - mu

    @pl.when(i == 0)
    def _():
        acc[...] = jnp.zeros((1, D), _F32)

    acc[...] = acc[...] + jnp.sum(d * d, axis=0, keepdims=True)
    var_out[...] = acc[...]


_var_call = pl.pallas_call(
    _var_body,
    grid=(NBLK,),
    in_specs=[pl.BlockSpec((BLK, D), lambda i: (i, 0)),
              pl.BlockSpec((1, D), lambda i: (0, 0))],
    out_specs=[pl.BlockSpec((1, D), lambda i: (0, 0))],
    out_shape=[jax.ShapeDtypeStruct((1, D), _F32)],
    scratch_shapes=[pltpu.VMEM((1, D), _F32)],
)


def _pool_body(hpre, stats, g, b, bat, l1W, l1b, l2W, l2b, out, seg, cnt):
    i = pl.program_id(0)
    mu = stats[0:1, :] * (1.0 / N)
    var = stats[1:2, :] * (1.0 / N)
    h = jnp.maximum((hpre[...] - mu) * lax.rsqrt(var + 1e-5) * g[...] + b[...],
                    0.0)
    ids = lax.broadcasted_iota(jnp.int32, (BLK, G), 1)
    oh = (ids == bat[...]).astype(_F32)

    @pl.when(i == 0)
    def _():
        seg[...] = jnp.zeros((G, D), _F32)
        cnt[...] = jnp.zeros((G, 1), _F32)

    seg[...] = seg[...] + lax.dot_general(
        oh, h, (((0,), (0,)), ((), ())), preferred_element_type=_F32)
    cnt[...] = cnt[...] + jnp.sum(oh, axis=0).reshape(G, 1)
    gf = seg[...] / jnp.maximum(cnt[...], 1.0)
    a1 = jnp.maximum(
        jnp.dot(gf, l1W[...], preferred_element_type=_F32) + l1b[...], 0.0)
    out[...] = (jnp.dot(a1, l2W[...], preferred_element_type=_F32)
                + l2b[...])


_pool_call = pl.pallas_call(
    _pool_body,
    grid=(NBLK,),
    in_specs=[pl.BlockSpec((BLK, D), lambda i: (i, 0)),
              pl.BlockSpec((2, D), lambda i: (0, 0)),
              pl.BlockSpec((1, D), lambda i: (0, 0)),
              pl.BlockSpec((1, D), lambda i: (0, 0)),
              pl.BlockSpec((BLK, 1), lambda i: (i, 0)),
              pl.BlockSpec((D, D), lambda i: (0, 0)),
              pl.BlockSpec((1, D), lambda i: (0, 0)),
              pl.BlockSpec((D, 1), lambda i: (0, 0)),
              pl.BlockSpec((1, 1), lambda i: (0, 0))],
    out_specs=[pl.BlockSpec((G, 1), lambda i: (0, 0))],
    out_shape=[jax.ShapeDtypeStruct((G, 1), _F32)],
    scratch_shapes=[pltpu.VMEM((G, D), _F32), pltpu.VMEM((G, 1), _F32)],
)


NP = 10112
_R = NP // NS
_RC = 104


def _edge_body(h_hbm, as_hbm, ad_hbm, M_hbm, src_hbm, dst_hbm,
               num_out, z_out,
               srcb, dstb, asb, adb, Mb, wba, wbb,
               rga, rgb, rsa, rsb, zbuf, zb, num_sh, z_sh,
               sga, sgb, ssa, ssb, sza, szb):
    cid = lax.axis_index("c")
    sid = lax.axis_index("s")

    pltpu.sync_copy(src_hbm.at[sid], srcb)
    pltpu.sync_copy(dst_hbm.at[sid], dstb)
    pltpu.sync_copy(as_hbm, asb)
    pltpu.sync_copy(ad_hbm, adb)
    pltpu.sync_copy(M_hbm, Mb)

    def _zrow(i, c):
        for q in range(DH // 16):
            zbuf[i, pl.ds(q * 16, 16)] = jnp.zeros((16,), _F32)
        return c

    lax.fori_loop(0, _RC, _zrow, 0)
    for k in range(_R // _RC):
        pltpu.sync_copy(zbuf, num_sh.at[pl.ds(sid * _R + k * _RC, _RC)])
    pltpu.sync_copy(zbuf.at[pl.ds(0, _R % _RC)],
                    num_sh.at[pl.ds(sid * _R + (_R // _RC) * _RC, _R % _RC)])

    def _zv(i, c):
        zb[pl.ds(i * 16, 16)] = jnp.zeros((16,), _F32)
        return c

    lax.fori_loop(0, 40, _zv, 0)
    pltpu.sync_copy(zb.at[pl.ds(0, _R)], z_sh.at[pl.ds(sid * _R, _R)])

    plsc.subcore_barrier()

    Mv = Mb[0, pl.ds(0, 16)]

    hsrc = h_hbm.at[cid]

    pltpu.async_copy(hsrc.at[srcb.at[0]], rga, sga)
    pltpu.async_copy(hsrc.at[srcb.at[1]], rgb, sgb)

    def _sub(k, bi, rg, rs, wb, sg, ss, sz):
        @pl.when(k > 0)
        def _():
            pltpu.make_async_copy(wb, z_sh.at[dstb.at[bi]], sz).wait()

        for gidx in range(BB // 16):
            srcv = srcb[bi, pl.ds(gidx * 16, 16)]
            dstv = dstb[bi, pl.ds(gidx * 16, 16)]
            e = plsc.load_gather(asb, [srcv]) + plsc.load_gather(adb, [dstv])
            e = jnp.where(e >= 0, e, 0.2 * e) - Mv
            wb[pl.ds(gidx * 16, 16)] = jnp.exp(e)

        @pl.when(k > 0)
        def _():
            pltpu.make_async_copy(rs, num_sh.at[dstb.at[bi]], ss).wait()

        pltpu.make_async_copy(hsrc.at[srcb.at[bi]], rg, sg).wait()

        def _srow(j, c2):
            wsp = plsc.load_gather(wb, [jnp.full((16,), j, jnp.int32)])
            for q in range(DH // 16):
                rs[j, pl.ds(q * 16, 16)] = rg[j, pl.ds(q * 16, 16)] * wsp
            return c2

        lax.fori_loop(0, BB, _srow, 0)

        @pl.when(k < NB // 2 - 1)
        def _():
            pltpu.async_copy(hsrc.at[srcb.at[bi + 2]], rg, sg)

        pltpu.async_copy(rs, num_sh.at[dstb.at[bi]], ss, add=True)
        pltpu.async_copy(wb, z_sh.at[dstb.at[bi]], sz, add=True)

    def _batch(k, c):
        _sub(k, 2 * k, rga, rsa, wba, sga, ssa, sza)
        _sub(k, 2 * k + 1, rgb, rsb, wbb, sgb, ssb, szb)
        return c

    lax.fori_loop(0, NB // 2, _batch, 0)

    pltpu.make_async_copy(rsa, num_sh.at[dstb.at[NB - 2]], ssa).wait()
    pltpu.make_async_copy(rsb, num_sh.at[dstb.at[NB - 1]], ssb).wait()
    pltpu.make_async_copy(wba, z_sh.at[dstb.at[NB - 2]], sza).wait()
    pltpu.make_async_copy(wbb, z_sh.at[dstb.at[NB - 1]], szb).wait()
    plsc.subcore_barrier()

    sl = pl.ds(sid * _R, _R)
    pltpu.sync_copy(num_sh.at[sl], num_out.at[cid].at[sl])
    pltpu.sync_copy(z_sh.at[sl], z_out.at[cid].at[sl])


_edge_call = pl.kernel(
    _edge_body,
    out_type=[jax.ShapeDtypeStruct((NC, NP, DH), _F32),
              jax.ShapeDtypeStruct((NC, NP), _F32)],
    mesh=plsc.VectorSubcoreMesh(core_axis_name="c", subcore_axis_name="s",
                                num_cores=NC, num_subcores=NS),
    scratch_types=[
        pltpu.VMEM((NB, BB), jnp.int32),
        pltpu.VMEM((NB, BB), jnp.int32),
        pltpu.VMEM((N,), _F32),
        pltpu.VMEM((N,), _F32),
        pltpu.VMEM((1, D), _F32),
        pltpu.VMEM((BB,), _F32),
        pltpu.VMEM((BB,), _F32),
        pltpu.VMEM((BB, DH), _F32),
        pltpu.VMEM((BB, DH), _F32),
        pltpu.VMEM((BB, DH), _F32),
        pltpu.VMEM((BB, DH), _F32),
        pltpu.VMEM((_RC, DH), _F32),
        pltpu.VMEM((640,), _F32),
        pltpu.VMEM_SHARED((NP, DH), _F32),
        pltpu.VMEM_SHARED((NP,), _F32),
        pltpu.SemaphoreType.DMA,
        pltpu.SemaphoreType.DMA,
        pltpu.SemaphoreType.DMA,
        pltpu.SemaphoreType.DMA,
        pltpu.SemaphoreType.DMA,
        pltpu.SemaphoreType.DMA,
    ],
    compiler_params=pltpu.CompilerParams(needs_layout_passes=False,
                                         use_tc_tiling_on_sc=False),
)


def kernel(x, edge_index, batch, params):
    src3 = edge_index[0].reshape(NS, NB, BB)
    dst3 = edge_index[1].reshape(NS, NB, BB)
    bat2 = batch.reshape(N, 1)

    def row(v):
        return v.reshape(1, D)

    def layer(hin, stats, bn_prev, gat):
        if stats is None:
            h, a_s, a_d, M = _prep0(hin, gat["W"], row(gat["a_src"]),
                                    row(gat["a_dst"]))
        else:
            h, a_s, a_d, M = _prep1(hin, stats, row(bn_prev["g"]),
                                    row(bn_prev["b"]), gat["W"],
                                    row(gat["a_src"]), row(gat["a_dst"]))
        num, z = _edge_call(h, a_s.reshape(N), a_d.reshape(N), M, src3, dst3)
        hpre, csum = _stats_call(num, z.reshape(NC, NP, 1), row(gat["b"]))
        (cvar,) = _var_call(hpre, csum)
        return hpre, jnp.concatenate([csum, cvar], axis=0)

    hpre, st = layer(x, None, None, params["gat1"])
    hpre, st = layer(hpre, st, params["bn1"], params["gat2"])
    hpre, st = layer(hpre, st, params["bn2"], params["gat3"])

    (out,) = _pool_call(hpre, st, row(params["bn3"]["g"]),
                        row(params["bn3"]["b"]), bat2,
                        params["lin1_W"], row(params["lin1_b"]),
                        params["lin2_W"], params["lin2_b"].reshape(1, 1))
    return out


# speedup vs baseline: 40.7894x; 2.1548x over previous
import functools

import jax
import jax.numpy as jnp
from jax import lax
from jax.experimental import pallas as pl
from jax.experimental.pallas import tpu as pltpu
from jax.experimental.pallas import tpu_sc as plsc

N = 10000
E = 320000
D = 128
G = 64

NC = 2
NS = 16
DH = D // NC
EPT = E // NS
BB = 80
NB = EPT // BB

BLK = 1000
NBLK = N // BLK

_F32 = jnp.float32


def _prep_body(use_bn, *refs):
    if use_bn:
        (hin, stats, g, b, W, asrc, adst, h_out, as_out, ad_out, M_out,
         msc) = refs
    else:
        hin, W, asrc, adst, h_out, as_out, ad_out, M_out, msc = refs
    i = pl.program_id(0)
    xb = hin[...]
    if use_bn:
        mu = stats[0:1, :] * (1.0 / N)
        var = stats[1:2, :] * (1.0 / N)
        xb = (xb - mu) * lax.rsqrt(var + 1e-5) * g[...] + b[...]
        xb = jnp.maximum(xb, 0.0)
    h = jnp.dot(xb, W[...], preferred_element_type=_F32)
    h_out[0] = h[:, :DH]
    h_out[1] = h[:, DH:]
    a_s = jnp.sum(h * asrc[...], axis=1, keepdims=True)
    a_d = jnp.sum(h * adst[...], axis=1, keepdims=True)
    as_out[...] = a_s
    ad_out[...] = a_d

    @pl.when(i == 0)
    def _():
        msc[0] = -jnp.inf
        msc[1] = -jnp.inf

    msc[0] = jnp.maximum(msc[0], jnp.max(a_s))
    msc[1] = jnp.maximum(msc[1], jnp.max(a_d))
    mm = msc[0] + msc[1]
    M = jnp.where(mm >= 0, mm, 0.2 * mm)
    M_out[...] = jnp.full((1, D), M, _F32)


def _make_prep(use_bn):
    full = lambda i: (0, 0)
    in_specs = [pl.BlockSpec((BLK, D), lambda i: (i, 0))]
    if use_bn:
        in_specs += [pl.BlockSpec((2, D), full), pl.BlockSpec((1, D), full),
                     pl.BlockSpec((1, D), full)]
    in_specs += [pl.BlockSpec((D, D), full), pl.BlockSpec((1, D), full),
                 pl.BlockSpec((1, D), full)]
    return pl.pallas_call(
        functools.partial(_prep_body, use_bn),
        grid=(NBLK,),
        in_specs=in_specs,
        out_specs=[pl.BlockSpec((NC, BLK, DH), lambda i: (0, i, 0)),
                   pl.BlockSpec((BLK, 1), lambda i: (i, 0)),
                   pl.BlockSpec((BLK, 1), lambda i: (i, 0)),
                   pl.BlockSpec((1, D), full)],
        out_shape=[jax.ShapeDtypeStruct((NC, N, DH), _F32),
                   jax.ShapeDtypeStruct((N, 1), _F32),
                   jax.ShapeDtypeStruct((N, 1), _F32),
                   jax.ShapeDtypeStruct((1, D), _F32)],
        scratch_shapes=[pltpu.SMEM((2,), _F32)],
    )


_prep0 = _make_prep(False)
_prep1 = _make_prep(True)


def _stats_body(num, z, b, hpre_out, sum_out, acc):
    i = pl.program_id(0)
    s = jnp.concatenate([num[0], num[1]], axis=1)
    zz = z[0]
    hp = s / (zz + 1e-16) + b[...]
    hpre_out[...] = hp

    @pl.when(i == 0)
    def _():
        acc[...] = jnp.zeros((1, D), _F32)

    acc[...] = acc[...] + jnp.sum(hp, axis=0, keepdims=True)
    sum_out[...] = acc[...]


_stats_call = pl.pallas_call(
    _stats_body,
    grid=(NBLK,),
    in_specs=[pl.BlockSpec((NC, BLK, DH), lambda i: (0, i, 0)),
              pl.BlockSpec((NC, BLK, 1), lambda i: (0, i, 0)),
              pl.BlockSpec((1, D), lambda i: (0, 0))],
    out_specs=[pl.BlockSpec((BLK, D), lambda i: (i, 0)),
               pl.BlockSpec((1, D), lambda i: (0, 0))],
    out_shape=[jax.ShapeDtypeStruct((N, D), _F32),
               jax.ShapeDtypeStruct((1, D), _F32)],
    scratch_shapes=[pltpu.VMEM((1, D), _F32)],
)


def _var_body(hpre, csum, var_out, acc):
    i = pl.program_id(0)
    mu = csum[...] * (1.0 / N)
    d = hpre[...] - mu

    @pl.when(i == 0)
    def _():
        acc[...] = jnp.zeros((1, D), _F32)

    acc[...] = acc[...] + jnp.sum(d * d, axis=0, keepdims=True)
    var_out[...] = acc[...]


_var_call = pl.pallas_call(
    _var_body,
    grid=(NBLK,),
    in_specs=[pl.BlockSpec((BLK, D), lambda i: (i, 0)),
              pl.BlockSpec((1, D), lambda i: (0, 0))],
    out_specs=[pl.BlockSpec((1, D), lambda i: (0, 0))],
    out_shape=[jax.ShapeDtypeStruct((1, D), _F32)],
    scratch_shapes=[pltpu.VMEM((1, D), _F32)],
)


def _pool_body(hpre, stats, g, b, bat, l1W, l1b, l2W, l2b, out, seg, cnt):
    i = pl.program_id(0)
    mu = stats[0:1, :] * (1.0 / N)
    var = stats[1:2, :] * (1.0 / N)
    h = jnp.maximum((hpre[...] - mu) * lax.rsqrt(var + 1e-5) * g[...] + b[...],
                    0.0)
    ids = lax.broadcasted_iota(jnp.int32, (BLK, G), 1)
    oh = (ids == bat[...]).astype(_F32)

    @pl.when(i == 0)
    def _():
        seg[...] = jnp.zeros((G, D), _F32)
        cnt[...] = jnp.zeros((G, 1), _F32)

    seg[...] = seg[...] + lax.dot_general(
        oh, h, (((0,), (0,)), ((), ())), preferred_element_type=_F32)
    cnt[...] = cnt[...] + jnp.sum(oh, axis=0).reshape(G, 1)
    gf = seg[...] / jnp.maximum(cnt[...], 1.0)
    a1 = jnp.maximum(
        jnp.dot(gf, l1W[...], preferred_element_type=_F32) + l1b[...], 0.0)
    out[...] = (jnp.dot(a1, l2W[...], preferred_element_type=_F32)
                + l2b[...])


_pool_call = pl.pallas_call(
    _pool_body,
    grid=(NBLK,),
    in_specs=[pl.BlockSpec((BLK, D), lambda i: (i, 0)),
              pl.BlockSpec((2, D), lambda i: (0, 0)),
              pl.BlockSpec((1, D), lambda i: (0, 0)),
              pl.BlockSpec((1, D), lambda i: (0, 0)),
              pl.BlockSpec((BLK, 1), lambda i: (i, 0)),
              pl.BlockSpec((D, D), lambda i: (0, 0)),
              pl.BlockSpec((1, D), lambda i: (0, 0)),
              pl.BlockSpec((D, 1), lambda i: (0, 0)),
              pl.BlockSpec((1, 1), lambda i: (0, 0))],
    out_specs=[pl.BlockSpec((G, 1), lambda i: (0, 0))],
    out_shape=[jax.ShapeDtypeStruct((G, 1), _F32)],
    scratch_shapes=[pltpu.VMEM((G, D), _F32), pltpu.VMEM((G, 1), _F32)],
)


NP = 10112
_R = NP // NS
_RC = 104


def _edge_body(h_hbm, as_hbm, ad_hbm, M_hbm, src_hbm, dst_hbm,
               num_out, z_out,
               srcb, dstb, asb, adb, Mb, wba, wbb,
               rga, rgb, rsa, rsb, zbuf, zb, num_sh, z_sh,
               sga, sgb, ssa, ssb, sza, szb):
    cid = lax.axis_index("c")
    sid = lax.axis_index("s")

    pltpu.sync_copy(src_hbm.at[sid], srcb)
    pltpu.sync_copy(dst_hbm.at[sid], dstb)
    pltpu.sync_copy(as_hbm, asb)
    pltpu.sync_copy(ad_hbm, adb)
    pltpu.sync_copy(M_hbm, Mb)

    def _zrow(i, c):
        for q in range(DH // 16):
            zbuf[i, pl.ds(q * 16, 16)] = jnp.zeros((16,), _F32)
        return c

    lax.fori_loop(0, _RC, _zrow, 0)
    for k in range(_R // _RC):
        pltpu.sync_copy(zbuf, num_sh.at[pl.ds(sid * _R + k * _RC, _RC)])
    pltpu.sync_copy(zbuf.at[pl.ds(0, _R % _RC)],
                    num_sh.at[pl.ds(sid * _R + (_R // _RC) * _RC, _R % _RC)])

    def _zv(i, c):
        zb[pl.ds(i * 16, 16)] = jnp.zeros((16,), _F32)
        return c

    lax.fori_loop(0, 40, _zv, 0)
    pltpu.sync_copy(zb.at[pl.ds(0, _R)], z_sh.at[pl.ds(sid * _R, _R)])

    plsc.subcore_barrier()

    Mv = Mb[0, pl.ds(0, 16)]

    hsrc = h_hbm.at[cid]

    pltpu.async_copy(hsrc.at[srcb.at[0]], rga, sga)
    pltpu.async_copy(hsrc.at[srcb.at[1]], rgb, sgb)

    def _sub(k, bi, rg, rs, wb, sg, ss, sz):
        @pl.when(k > 0)
        def _():
            pltpu.make_async_copy(wb, z_sh.at[dstb.at[bi]], sz).wait()

        for gidx in range(BB // 16):
            srcv = srcb[bi, pl.ds(gidx * 16, 16)]
            dstv = dstb[bi, pl.ds(gidx * 16, 16)]
            e = plsc.load_gather(asb, [srcv]) + plsc.load_gather(adb, [dstv])
            e = jnp.where(e >= 0, e, 0.2 * e) - Mv
            wb[pl.ds(gidx * 16, 16)] = jnp.exp(e)

        @pl.when(k > 0)
        def _():
            pltpu.make_async_copy(rs, num_sh.at[dstb.at[bi]], ss).wait()

        pltpu.async_copy(wb, z_sh.at[dstb.at[bi]], sz, add=True)

        pltpu.make_async_copy(hsrc.at[srcb.at[bi]], rg, sg).wait()

        def _srow16(g2, c2):
            wv = wb[pl.ds(g2 * 16, 16)]
            for l in range(16):
                j = g2 * 16 + l
                wsp = jnp.full((16,), wv[l], _F32)
                for q in range(DH // 16):
                    rs[j, pl.ds(q * 16, 16)] = rg[j, pl.ds(q * 16, 16)] * wsp
            return c2

        lax.fori_loop(0, BB // 16, _srow16, 0)

        @pl.when(k < NB // 2 - 1)
        def _():
            pltpu.async_copy(hsrc.at[srcb.at[bi + 2]], rg, sg)

        pltpu.async_copy(rs, num_sh.at[dstb.at[bi]], ss, add=True)

    def _batch(k, c):
        _sub(k, 2 * k, rga, rsa, wba, sga, ssa, sza)
        _sub(k, 2 * k + 1, rgb, rsb, wbb, sgb, ssb, szb)
        return c

    lax.fori_loop(0, NB // 2, _batch, 0)

    pltpu.make_async_copy(rsa, num_sh.at[dstb.at[NB - 2]], ssa).wait()
    pltpu.make_async_copy(rsb, num_sh.at[dstb.at[NB - 1]], ssb).wait()
    pltpu.make_async_copy(wba, z_sh.at[dstb.at[NB - 2]], sza).wait()
    pltpu.make_async_copy(wbb, z_sh.at[dstb.at[NB - 1]], szb).wait()
    plsc.subcore_barrier()

    sl = pl.ds(sid * _R, _R)
    pltpu.sync_copy(num_sh.at[sl], num_out.at[cid].at[sl])
    pltpu.sync_copy(z_sh.at[sl], z_out.at[cid].at[sl])


_edge_call = pl.kernel(
    _edge_body,
    out_type=[jax.ShapeDtypeStruct((NC, NP, DH), _F32),
              jax.ShapeDtypeStruct((NC, NP), _F32)],
    mesh=plsc.VectorSubcoreMesh(core_axis_name="c", subcore_axis_name="s",
                                num_cores=NC, num_subcores=NS),
    scratch_types=[
        pltpu.VMEM((NB, BB), jnp.int32),
        pltpu.VMEM((NB, BB), jnp.int32),
        pltpu.VMEM((N,), _F32),
        pltpu.VMEM((N,), _F32),
        pltpu.VMEM((1, D), _F32),
        pltpu.VMEM((BB,), _F32),
        pltpu.VMEM((BB,), _F32),
        pltpu.VMEM((BB, DH), _F32),
        pltpu.VMEM((BB, DH), _F32),
        pltpu.VMEM((BB, DH), _F32),
        pltpu.VMEM((BB, DH), _F32),
        pltpu.VMEM((_RC, DH), _F32),
        pltpu.VMEM((640,), _F32),
        pltpu.VMEM_SHARED((NP, DH), _F32),
        pltpu.VMEM_SHARED((NP,), _F32),
        pltpu.SemaphoreType.DMA,
        pltpu.SemaphoreType.DMA,
        pltpu.SemaphoreType.DMA,
        pltpu.SemaphoreType.DMA,
        pltpu.SemaphoreType.DMA,
        pltpu.SemaphoreType.DMA,
    ],
    compiler_params=pltpu.CompilerParams(needs_layout_passes=False,
                                         use_tc_tiling_on_sc=False),
)


def kernel(x, edge_index, batch, params):
    src3 = edge_index[0].reshape(NS, NB, BB)
    dst3 = edge_index[1].reshape(NS, NB, BB)
    bat2 = batch.reshape(N, 1)

    def row(v):
        return v.reshape(1, D)

    def layer(hin, stats, bn_prev, gat):
        if stats is None:
            h, a_s, a_d, M = _prep0(hin, gat["W"], row(gat["a_src"]),
                                    row(gat["a_dst"]))
        else:
            h, a_s, a_d, M = _prep1(hin, stats, row(bn_prev["g"]),
                                    row(bn_prev["b"]), gat["W"],
                                    row(gat["a_src"]), row(gat["a_dst"]))
        num, z = _edge_call(h, a_s.reshape(N), a_d.reshape(N), M, src3, dst3)
        hpre, csum = _stats_call(num, z.reshape(NC, NP, 1), row(gat["b"]))
        (cvar,) = _var_call(hpre, csum)
        return hpre, jnp.concatenate([csum, cvar], axis=0)

    hpre, st = layer(x, None, None, params["gat1"])
    hpre, st = layer(hpre, st, params["bn1"], params["gat2"])
    hpre, st = layer(hpre, st, params["bn2"], params["gat3"])

    (out,) = _pool_call(hpre, st, row(params["bn3"]["g"]),
                        row(params["bn3"]["b"]), bat2,
                        params["lin1_W"], row(params["lin1_b"]),
                        params["lin2_W"], params["lin2_b"].reshape(1, 1))
    return out
